# Initial kernel scaffold; baseline (speedup 1.0000x reference)
import jax
import jax.numpy as jnp
from jax.experimental import pallas as pl


def kernel(x, edge_index, W, b):
    raise NotImplementedError("write your pallas kernel here")


# speedup vs baseline: 31.2715x; 31.2715x over previous
import functools

import jax
import jax.numpy as jnp
from jax import lax
from jax.experimental import pallas as pl
from jax.experimental.pallas import tpu as pltpu
from jax.experimental.pallas import tpu_sc as plsc

NC = 2
NS = 16
NW = NC * NS
CHUNK = 128


def _sc_mesh():
    return plsc.VectorSubcoreMesh(
        core_axis_name="c", subcore_axis_name="s", num_cores=NC,
        num_subcores=NS)


def _make_hist(n_pad, cpt):
    @functools.partial(
        pl.kernel,
        out_type=jax.ShapeDtypeStruct((NW, n_pad), jnp.float32),
        mesh=_sc_mesh(),
        compiler_params=pltpu.CompilerParams(needs_layout_passes=False),
        scratch_types=[
            pltpu.VMEM((cpt, CHUNK), jnp.int32),
            pltpu.VMEM((n_pad,), jnp.float32),
        ],
    )
    def hist_kernel(dst_hbm, hist_out, dst_v, loc_v):
        cid = lax.axis_index("c")
        sid = lax.axis_index("s")
        wid = sid * NC + cid

        def zstep(i, carry):
            loc_v[pl.ds(i * 16, 16)] = jnp.zeros((16,), jnp.float32)
            return carry

        lax.fori_loop(0, n_pad // 16, zstep, 0)
        pltpu.sync_copy(dst_hbm.at[wid], dst_v)
        ones = jnp.ones((16,), jnp.float32)

        def step(j, carry):
            def inner(i, carry2):
                idx = dst_v[j, pl.ds(i * 16, 16)]
                plsc.addupdate_scatter(loc_v, [idx], ones)
                return carry2

            return lax.fori_loop(0, CHUNK // 16, inner, carry)

        lax.fori_loop(0, cpt, step, 0)
        pltpu.sync_copy(loc_v, hist_out.at[wid])

    return hist_kernel


def _make_spmm(n_pad, cpt, d):
    rpt = n_pad // NS

    @functools.partial(
        pl.kernel,
        out_type=jax.ShapeDtypeStruct((NC, n_pad, d), jnp.float32),
        mesh=_sc_mesh(),
        scratch_types=[
            pltpu.VMEM((cpt, CHUNK), jnp.int32),
            pltpu.VMEM((cpt, CHUNK), jnp.int32),
            pltpu.VMEM((CHUNK, d), jnp.float32),
            pltpu.VMEM_SHARED((n_pad, d), jnp.float32),
            pltpu.SemaphoreType.DMA,
        ],
    )
    def spmm_kernel(g_hbm, src_hbm, dst_hbm, zrow_hbm, acc_out, src_v, dst_v,
                    rows_v, acc_sh, sem):
        cid = lax.axis_index("c")
        sid = lax.axis_index("s")
        wid = sid * NC + cid
        pltpu.sync_copy(src_hbm.at[wid], src_v)
        pltpu.sync_copy(dst_hbm.at[wid], dst_v)

        @pl.when(cid == 0)
        def _():
            pltpu.sync_copy(g_hbm.at[pl.ds(sid * rpt, rpt)],
                            acc_sh.at[pl.ds(sid * rpt, rpt)])

        @pl.when(cid == 1)
        def _():
            pltpu.sync_copy(zrow_hbm, acc_sh.at[pl.ds(sid * rpt, rpt)])

        plsc.subcore_barrier()

        def step(j, carry):
            pltpu.async_copy(g_hbm.at[src_v.at[j]], rows_v, sem).wait()
            pltpu.sync_copy(rows_v, acc_sh.at[dst_v.at[j]], add=True)
            return carry

        lax.fori_loop(0, cpt, step, 0)
        plsc.subcore_barrier()
        pltpu.sync_copy(acc_sh.at[pl.ds(sid * rpt, rpt)],
                        acc_out.at[cid].at[pl.ds(sid * rpt, rpt)])

    return spmm_kernel


def _linear_body(x_ref, w_ref, hist_ref, g_ref):
    h = jnp.dot(x_ref[...], w_ref[...], preferred_element_type=jnp.float32)
    deg = jnp.sum(hist_ref[...], axis=1, keepdims=True) + 1.0
    g_ref[...] = h * lax.rsqrt(deg)


def _final_body(acc_ref, hist_ref, b_ref, out_ref):
    deg = jnp.sum(hist_ref[...], axis=1, keepdims=True) + 1.0
    s = lax.rsqrt(deg)
    out_ref[...] = (acc_ref[0] + acc_ref[1]) * s + b_ref[...]


def kernel(x, edge_index, W, b):
    n, d_in = x.shape
    d = W.shape[1]
    e = edge_index.shape[1]

    n_pad = (n // 128 + 1) * 128
    n_pad = ((n_pad + 2047) // 2048) * 2048
    e_pad = ((e + NW * CHUNK - 1) // (NW * CHUNK)) * (NW * CHUNK)
    cpt = e_pad // (NW * CHUNK)
    rpt = n_pad // NS
    pad_cnt = e_pad - e
    pad_rows = n_pad - n

    pad_idx = (n + jnp.arange(pad_cnt, dtype=jnp.int32) % pad_rows)
    src_p = jnp.concatenate([edge_index[0], pad_idx]).reshape(NW, cpt, CHUNK)
    dst_p = jnp.concatenate([edge_index[1], pad_idx]).reshape(NW, cpt, CHUNK)
    x_p = jnp.pad(x, ((0, n_pad - n), (0, 0)))

    zrow_d = jnp.zeros((rpt, d), jnp.float32)

    hist = _make_hist(n_pad, cpt)(dst_p).T

    bt = 1280
    grid = (n_pad // bt,)
    g = pl.pallas_call(
        _linear_body,
        grid=grid,
        in_specs=[
            pl.BlockSpec((bt, d_in), lambda i: (i, 0)),
            pl.BlockSpec((d_in, d), lambda i: (0, 0)),
            pl.BlockSpec((bt, NW), lambda i: (i, 0)),
        ],
        out_specs=pl.BlockSpec((bt, d), lambda i: (i, 0)),
        out_shape=jax.ShapeDtypeStruct((n_pad, d), jnp.float32),
    )(x_p, W, hist)

    acc = _make_spmm(n_pad, cpt, d)(g, src_p, dst_p, zrow_d)

    out = pl.pallas_call(
        _final_body,
        grid=grid,
        in_specs=[
            pl.BlockSpec((NC, bt, d), lambda i: (0, i, 0)),
            pl.BlockSpec((bt, NW), lambda i: (i, 0)),
            pl.BlockSpec((1, d), lambda i: (0, 0)),
        ],
        out_specs=pl.BlockSpec((bt, d), lambda i: (i, 0)),
        out_shape=jax.ShapeDtypeStruct((n_pad, d), jnp.float32),
    )(acc, hist, b.reshape(1, d))

    return out[:n]


# speedup vs baseline: 37.5222x; 1.1999x over previous
import functools

import jax
import jax.numpy as jnp
from jax import lax
from jax.experimental import pallas as pl
from jax.experimental.pallas import tpu as pltpu
from jax.experimental.pallas import tpu_sc as plsc

NC = 2
NS = 16
NW = NC * NS
CHUNK = 128


def _sc_mesh():
    return plsc.VectorSubcoreMesh(
        core_axis_name="c", subcore_axis_name="s", num_cores=NC,
        num_subcores=NS)


def _make_hist(n_pad, cpt):
    @functools.partial(
        pl.kernel,
        out_type=jax.ShapeDtypeStruct((NW, n_pad), jnp.float32),
        mesh=_sc_mesh(),
        compiler_params=pltpu.CompilerParams(needs_layout_passes=False),
        scratch_types=[
            pltpu.VMEM((cpt, CHUNK), jnp.int32),
            pltpu.VMEM((n_pad,), jnp.float32),
        ],
    )
    def hist_kernel(dst_hbm, hist_out, dst_v, loc_v):
        cid = lax.axis_index("c")
        sid = lax.axis_index("s")
        wid = sid * NC + cid

        def zstep(i, carry):
            loc_v[pl.ds(i * 16, 16)] = jnp.zeros((16,), jnp.float32)
            return carry

        lax.fori_loop(0, n_pad // 16, zstep, 0)
        pltpu.sync_copy(dst_hbm.at[wid], dst_v)
        ones = jnp.ones((16,), jnp.float32)

        def step(j, carry):
            def inner(i, carry2):
                idx = dst_v[j, pl.ds(i * 16, 16)]
                plsc.addupdate_scatter(loc_v, [idx], ones)
                return carry2

            return lax.fori_loop(0, CHUNK // 16, inner, carry)

        lax.fori_loop(0, cpt, step, 0)
        pltpu.sync_copy(loc_v, hist_out.at[wid])

    return hist_kernel


def _make_spmm(n_pad, cpt, d):
    rpt = n_pad // NS

    @functools.partial(
        pl.kernel,
        out_type=jax.ShapeDtypeStruct((NC, n_pad, d), jnp.float32),
        mesh=_sc_mesh(),
        scratch_types=[
            pltpu.VMEM((2, CHUNK), jnp.int32),
            pltpu.VMEM((2, CHUNK), jnp.int32),
            pltpu.VMEM((CHUNK, d), jnp.float32),
            pltpu.VMEM((CHUNK, d), jnp.float32),
            pltpu.VMEM_SHARED((n_pad, d), jnp.float32),
            pltpu.SemaphoreType.DMA,
            pltpu.SemaphoreType.DMA,
            pltpu.SemaphoreType.DMA,
            pltpu.SemaphoreType.DMA,
        ],
    )
    def spmm_kernel(g_hbm, edges_hbm, zrow_hbm, acc_out, win0_v, win1_v,
                    rows0_v, rows1_v, acc_sh, semg0, semg1, semi0, semi1):
        cid = lax.axis_index("c")
        sid = lax.axis_index("s")
        wid = sid * NC + cid

        @pl.when(cid == 0)
        def _():
            pltpu.sync_copy(g_hbm.at[pl.ds(sid * rpt, rpt)],
                            acc_sh.at[pl.ds(sid * rpt, rpt)])

        @pl.when(cid == 1)
        def _():
            pltpu.sync_copy(zrow_hbm, acc_sh.at[pl.ds(sid * rpt, rpt)])

        plsc.subcore_barrier()

        def wait_rows(buf, sem):
            pltpu.make_async_copy(g_hbm.at[pl.ds(0, CHUNK)], buf, sem).wait()

        def wait_win(buf, sem):
            pltpu.make_async_copy(edges_hbm.at[0].at[0], buf, sem).wait()

        pltpu.sync_copy(edges_hbm.at[wid].at[0], win0_v)
        pltpu.async_copy(g_hbm.at[win0_v.at[0]], rows0_v, semg0)

        @pl.when(1 < cpt)
        def _():
            pltpu.async_copy(edges_hbm.at[wid].at[1], win1_v, semi1)

        def step(k, carry):
            j1 = 2 * k + 1
            j2 = j1 + 1
            j3 = j1 + 2

            @pl.when(j1 < cpt)
            def _():
                wait_win(win1_v, semi1)
                pltpu.async_copy(g_hbm.at[win1_v.at[0]], rows1_v, semg1)

            wait_rows(rows0_v, semg0)
            pltpu.sync_copy(rows0_v, acc_sh.at[win0_v.at[1]], add=True)

            @pl.when(j2 < cpt)
            def _():
                pltpu.async_copy(edges_hbm.at[wid].at[j2], win0_v, semi0)

            @pl.when(j1 < cpt)
            def _():
                wait_rows(rows1_v, semg1)
                pltpu.sync_copy(rows1_v, acc_sh.at[win1_v.at[1]], add=True)

            @pl.when(j2 < cpt)
            def _():
                wait_win(win0_v, semi0)
                pltpu.async_copy(g_hbm.at[win0_v.at[0]], rows0_v, semg0)

            @pl.when(j3 < cpt)
            def _():
                pltpu.async_copy(edges_hbm.at[wid].at[j3], win1_v, semi1)

            return carry

        lax.fori_loop(0, (cpt + 1) // 2, step, 0)
        plsc.subcore_barrier()
        pltpu.sync_copy(acc_sh.at[pl.ds(sid * rpt, rpt)],
                        acc_out.at[cid].at[pl.ds(sid * rpt, rpt)])

    return spmm_kernel


def _linear_body(x_ref, w_ref, hist_ref, g_ref):
    h = jnp.dot(x_ref[...], w_ref[...], preferred_element_type=jnp.float32)
    deg = jnp.sum(hist_ref[...], axis=1, keepdims=True) + 1.0
    g_ref[...] = h * lax.rsqrt(deg)


def _final_body(acc_ref, hist_ref, b_ref, out_ref):
    deg = jnp.sum(hist_ref[...], axis=1, keepdims=True) + 1.0
    s = lax.rsqrt(deg)
    out_ref[...] = (acc_ref[0] + acc_ref[1]) * s + b_ref[...]


def kernel(x, edge_index, W, b):
    n, d_in = x.shape
    d = W.shape[1]
    e = edge_index.shape[1]

    n_pad = (n // 128 + 1) * 128
    n_pad = ((n_pad + 2047) // 2048) * 2048
    e_pad = ((e + NW * CHUNK - 1) // (NW * CHUNK)) * (NW * CHUNK)
    cpt = e_pad // (NW * CHUNK)
    rpt = n_pad // NS
    pad_cnt = e_pad - e
    pad_rows = n_pad - n

    pad_idx = (n + jnp.arange(pad_cnt, dtype=jnp.int32) % pad_rows)
    src_p = jnp.concatenate([edge_index[0], pad_idx]).reshape(NW, cpt, CHUNK)
    dst_p = jnp.concatenate([edge_index[1], pad_idx]).reshape(NW, cpt, CHUNK)
    edges_p = jnp.stack([src_p, dst_p], axis=2)
    x_p = jnp.pad(x, ((0, n_pad - n), (0, 0)))

    hist = _make_hist(n_pad, cpt)(dst_p).T

    zrow_d = jnp.zeros((rpt, d), jnp.float32)
    bt = 1280
    grid = (n_pad // bt,)
    g = pl.pallas_call(
        _linear_body,
        grid=grid,
        in_specs=[
            pl.BlockSpec((bt, d_in), lambda i: (i, 0)),
            pl.BlockSpec((d_in, d), lambda i: (0, 0)),
            pl.BlockSpec((bt, NW), lambda i: (i, 0)),
        ],
        out_specs=pl.BlockSpec((bt, d), lambda i: (i, 0)),
        out_shape=jax.ShapeDtypeStruct((n_pad, d), jnp.float32),
    )(x_p, W, hist)

    acc = _make_spmm(n_pad, cpt, d)(g, edges_p, zrow_d)

    out = pl.pallas_call(
        _final_body,
        grid=grid,
        in_specs=[
            pl.BlockSpec((NC, bt, d), lambda i: (0, i, 0)),
            pl.BlockSpec((bt, NW), lambda i: (i, 0)),
            pl.BlockSpec((1, d), lambda i: (0, 0)),
        ],
        out_specs=pl.BlockSpec((bt, d), lambda i: (i, 0)),
        out_shape=jax.ShapeDtypeStruct((n_pad, d), jnp.float32),
    )(acc, hist, b.reshape(1, d))

    return out[:n]


# speedup vs baseline: 41.6336x; 1.1096x over previous
import functools

import jax
import jax.numpy as jnp
from jax import lax
from jax.experimental import pallas as pl
from jax.experimental.pallas import tpu as pltpu
from jax.experimental.pallas import tpu_sc as plsc

NC = 2
NS = 16
NW = NC * NS
CHUNK = 128


def _sc_mesh():
    return plsc.VectorSubcoreMesh(
        core_axis_name="c", subcore_axis_name="s", num_cores=NC,
        num_subcores=NS)


def _make_hist(n_pad, cpt):
    @functools.partial(
        pl.kernel,
        out_type=jax.ShapeDtypeStruct((NW, n_pad), jnp.float32),
        mesh=_sc_mesh(),
        compiler_params=pltpu.CompilerParams(needs_layout_passes=False),
        scratch_types=[
            pltpu.VMEM((cpt, CHUNK), jnp.int32),
            pltpu.VMEM((n_pad,), jnp.float32),
        ],
    )
    def hist_kernel(dst_hbm, hist_out, dst_v, loc_v):
        cid = lax.axis_index("c")
        sid = lax.axis_index("s")
        wid = sid * NC + cid

        def zstep(i, carry):
            loc_v[pl.ds(i * 16, 16)] = jnp.zeros((16,), jnp.float32)
            return carry

        lax.fori_loop(0, n_pad // 16, zstep, 0)
        pltpu.sync_copy(dst_hbm.at[wid], dst_v)
        ones = jnp.ones((16,), jnp.float32)

        def step(j, carry):
            def inner(i, carry2):
                idx = dst_v[j, pl.ds(i * 16, 16)]
                plsc.addupdate_scatter(loc_v, [idx], ones)
                return carry2

            return lax.fori_loop(0, CHUNK // 16, inner, carry)

        lax.fori_loop(0, cpt, step, 0)
        pltpu.sync_copy(loc_v, hist_out.at[wid])

    return hist_kernel


def _make_spmm(n_pad, cpt, d):
    rpt = n_pad // NS

    @functools.partial(
        pl.kernel,
        out_type=jax.ShapeDtypeStruct((NC, n_pad, d), jnp.float32),
        mesh=_sc_mesh(),
        scratch_types=[
            pltpu.VMEM((2, CHUNK), jnp.int32),
            pltpu.VMEM((2, CHUNK), jnp.int32),
            pltpu.VMEM((CHUNK, d), jnp.float32),
            pltpu.VMEM((CHUNK, d), jnp.float32),
            pltpu.VMEM_SHARED((n_pad, d), jnp.float32),
            pltpu.SemaphoreType.DMA,
            pltpu.SemaphoreType.DMA,
            pltpu.SemaphoreType.DMA,
            pltpu.SemaphoreType.DMA,
        ],
    )
    def spmm_kernel(g_hbm, edges_hbm, zrow_hbm, acc_out, win0_v, win1_v,
                    rows0_v, rows1_v, acc_sh, semg0, semg1, semi0, semi1):
        cid = lax.axis_index("c")
        sid = lax.axis_index("s")
        wid = sid * NC + cid

        @pl.when(cid == 0)
        def _():
            pltpu.sync_copy(g_hbm.at[pl.ds(sid * rpt, rpt)],
                            acc_sh.at[pl.ds(sid * rpt, rpt)])

        @pl.when(cid == 1)
        def _():
            pltpu.sync_copy(zrow_hbm, acc_sh.at[pl.ds(sid * rpt, rpt)])

        plsc.subcore_barrier()

        def wait_rows(buf, sem):
            pltpu.make_async_copy(g_hbm.at[pl.ds(0, CHUNK)], buf, sem).wait()

        def wait_win(buf, sem):
            pltpu.make_async_copy(edges_hbm.at[0].at[0], buf, sem).wait()

        pltpu.sync_copy(edges_hbm.at[wid].at[0], win0_v)
        pltpu.async_copy(g_hbm.at[win0_v.at[0]], rows0_v, semg0)

        @pl.when(1 < cpt)
        def _():
            pltpu.async_copy(edges_hbm.at[wid].at[1], win1_v, semi1)

        def step(k, carry):
            j1 = 2 * k + 1
            j2 = j1 + 1
            j3 = j1 + 2

            @pl.when(j1 < cpt)
            def _():
                wait_win(win1_v, semi1)
                pltpu.async_copy(g_hbm.at[win1_v.at[0]], rows1_v, semg1)

            wait_rows(rows0_v, semg0)
            pltpu.sync_copy(rows0_v, acc_sh.at[win0_v.at[1]], add=True)

            @pl.when(j2 < cpt)
            def _():
                pltpu.async_copy(edges_hbm.at[wid].at[j2], win0_v, semi0)
                wait_win(win0_v, semi0)
                pltpu.async_copy(g_hbm.at[win0_v.at[0]], rows0_v, semg0)

            @pl.when(j1 < cpt)
            def _():
                wait_rows(rows1_v, semg1)
                pltpu.sync_copy(rows1_v, acc_sh.at[win1_v.at[1]], add=True)

            @pl.when(j3 < cpt)
            def _():
                pltpu.async_copy(edges_hbm.at[wid].at[j3], win1_v, semi1)

            return carry

        lax.fori_loop(0, (cpt + 1) // 2, step, 0)
        plsc.subcore_barrier()
        pltpu.sync_copy(acc_sh.at[pl.ds(sid * rpt, rpt)],
                        acc_out.at[cid].at[pl.ds(sid * rpt, rpt)])

    return spmm_kernel


def _linear_body(x_ref, w_ref, hist_ref, g_ref):
    h = jnp.dot(x_ref[...], w_ref[...], preferred_element_type=jnp.float32)
    deg = jnp.sum(hist_ref[...], axis=1, keepdims=True) + 1.0
    g_ref[...] = h * lax.rsqrt(deg)


def _final_body(acc_ref, hist_ref, b_ref, out_ref):
    deg = jnp.sum(hist_ref[...], axis=1, keepdims=True) + 1.0
    s = lax.rsqrt(deg)
    out_ref[...] = (acc_ref[0] + acc_ref[1]) * s + b_ref[...]


def kernel(x, edge_index, W, b):
    n, d_in = x.shape
    d = W.shape[1]
    e = edge_index.shape[1]

    n_pad = (n // 128 + 1) * 128
    n_pad = ((n_pad + 2047) // 2048) * 2048
    e_pad = ((e + NW * CHUNK - 1) // (NW * CHUNK)) * (NW * CHUNK)
    cpt = e_pad // (NW * CHUNK)
    rpt = n_pad // NS
    pad_cnt = e_pad - e
    pad_rows = n_pad - n

    pad_idx = (n + jnp.arange(pad_cnt, dtype=jnp.int32) % pad_rows)
    src_p = jnp.concatenate([edge_index[0], pad_idx]).reshape(NW, cpt, CHUNK)
    dst_p = jnp.concatenate([edge_index[1], pad_idx]).reshape(NW, cpt, CHUNK)
    edges_p = jnp.stack([src_p, dst_p], axis=2)

    hist = _make_hist(n_pad, cpt)(dst_p).T

    zrow_d = jnp.zeros((rpt, d), jnp.float32)
    bt = 2000
    grid = (n // bt,)
    g = pl.pallas_call(
        _linear_body,
        grid=grid,
        in_specs=[
            pl.BlockSpec((bt, d_in), lambda i: (i, 0)),
            pl.BlockSpec((d_in, d), lambda i: (0, 0)),
            pl.BlockSpec((bt, NW), lambda i: (i, 0)),
        ],
        out_specs=pl.BlockSpec((bt, d), lambda i: (i, 0)),
        out_shape=jax.ShapeDtypeStruct((n_pad, d), jnp.float32),
    )(x, W, hist)

    acc = _make_spmm(n_pad, cpt, d)(g, edges_p, zrow_d)

    out = pl.pallas_call(
        _final_body,
        grid=grid,
        in_specs=[
            pl.BlockSpec((NC, bt, d), lambda i: (0, i, 0)),
            pl.BlockSpec((bt, NW), lambda i: (i, 0)),
            pl.BlockSpec((1, d), lambda i: (0, 0)),
        ],
        out_specs=pl.BlockSpec((bt, d), lambda i: (i, 0)),
        out_shape=jax.ShapeDtypeStruct((n, d), jnp.float32),
    )(acc, hist, b.reshape(1, d))

    return out


# speedup vs baseline: 43.6511x; 1.0485x over previous
import functools

import jax
import jax.numpy as jnp
from jax import lax
from jax.experimental import pallas as pl
from jax.experimental.pallas import tpu as pltpu
from jax.experimental.pallas import tpu_sc as plsc

NC = 2
NS = 16
NW = NC * NS
CHUNK = 128


def _sc_mesh():
    return plsc.VectorSubcoreMesh(
        core_axis_name="c", subcore_axis_name="s", num_cores=NC,
        num_subcores=NS)


def _make_hist(n_pad, cpt):
    @functools.partial(
        pl.kernel,
        out_type=jax.ShapeDtypeStruct((NW, n_pad), jnp.float32),
        mesh=_sc_mesh(),
        compiler_params=pltpu.CompilerParams(needs_layout_passes=False),
        scratch_types=[
            pltpu.VMEM((cpt, 2, CHUNK), jnp.int32),
            pltpu.VMEM((n_pad,), jnp.float32),
        ],
    )
    def hist_kernel(edges_hbm, hist_out, ed_v, loc_v):
        cid = lax.axis_index("c")
        sid = lax.axis_index("s")
        wid = sid * NC + cid

        def zstep(i, carry):
            loc_v[pl.ds(i * 16, 16)] = jnp.zeros((16,), jnp.float32)
            return carry

        lax.fori_loop(0, n_pad // 16, zstep, 0)
        pltpu.sync_copy(edges_hbm.at[wid], ed_v)
        ones = jnp.ones((16,), jnp.float32)

        def step(j, carry):
            def inner(i, carry2):
                idx = ed_v[j, 1, pl.ds(i * 16, 16)]
                plsc.addupdate_scatter(loc_v, [idx], ones)
                return carry2

            return lax.fori_loop(0, CHUNK // 16, inner, carry)

        lax.fori_loop(0, cpt, step, 0)
        pltpu.sync_copy(loc_v, hist_out.at[wid])

    return hist_kernel


def _make_spmm(n_pad, cpt, d):
    rpt = n_pad // NS

    @functools.partial(
        pl.kernel,
        out_type=jax.ShapeDtypeStruct((NC, n_pad, d), jnp.float32),
        mesh=_sc_mesh(),
        scratch_types=[
            pltpu.VMEM((2, CHUNK), jnp.int32),
            pltpu.VMEM((2, CHUNK), jnp.int32),
            pltpu.VMEM((CHUNK, d), jnp.float32),
            pltpu.VMEM((CHUNK, d), jnp.float32),
            pltpu.VMEM_SHARED((n_pad, d), jnp.float32),
            pltpu.SemaphoreType.DMA,
            pltpu.SemaphoreType.DMA,
            pltpu.SemaphoreType.DMA,
            pltpu.SemaphoreType.DMA,
        ],
    )
    def spmm_kernel(g_hbm, edges_hbm, zrow_hbm, acc_out, win0_v, win1_v,
                    rows0_v, rows1_v, acc_sh, semg0, semg1, semi0, semi1):
        cid = lax.axis_index("c")
        sid = lax.axis_index("s")
        wid = sid * NC + cid

        pltpu.sync_copy(edges_hbm.at[wid].at[0], win0_v)
        pltpu.async_copy(g_hbm.at[win0_v.at[0]], rows0_v, semg0)

        @pl.when(1 < cpt)
        def _():
            pltpu.async_copy(edges_hbm.at[wid].at[1], win1_v, semi1)

        @pl.when(cid == 0)
        def _():
            pltpu.sync_copy(g_hbm.at[pl.ds(sid * rpt, rpt)],
                            acc_sh.at[pl.ds(sid * rpt, rpt)])

        @pl.when(cid == 1)
        def _():
            pltpu.sync_copy(zrow_hbm, acc_sh.at[pl.ds(sid * rpt, rpt)])

        plsc.subcore_barrier()

        def wait_rows(buf, sem):
            pltpu.make_async_copy(g_hbm.at[pl.ds(0, CHUNK)], buf, sem).wait()

        def wait_win(buf, sem):
            pltpu.make_async_copy(edges_hbm.at[0].at[0], buf, sem).wait()

        def step(k, carry):
            j1 = 2 * k + 1
            j2 = j1 + 1
            j3 = j1 + 2

            @pl.when(j1 < cpt)
            def _():
                wait_win(win1_v, semi1)
                pltpu.async_copy(g_hbm.at[win1_v.at[0]], rows1_v, semg1)

            wait_rows(rows0_v, semg0)
            pltpu.sync_copy(rows0_v, acc_sh.at[win0_v.at[1]], add=True)

            @pl.when(j2 < cpt)
            def _():
                pltpu.async_copy(edges_hbm.at[wid].at[j2], win0_v, semi0)
                wait_win(win0_v, semi0)
                pltpu.async_copy(g_hbm.at[win0_v.at[0]], rows0_v, semg0)

            @pl.when(j1 < cpt)
            def _():
                wait_rows(rows1_v, semg1)
                pltpu.sync_copy(rows1_v, acc_sh.at[win1_v.at[1]], add=True)

            @pl.when(j3 < cpt)
            def _():
                pltpu.async_copy(edges_hbm.at[wid].at[j3], win1_v, semi1)

            return carry

        lax.fori_loop(0, (cpt + 1) // 2, step, 0)
        plsc.subcore_barrier()
        pltpu.sync_copy(acc_sh.at[pl.ds(sid * rpt, rpt)],
                        acc_out.at[cid].at[pl.ds(sid * rpt, rpt)])

    return spmm_kernel


def _matmul_body(x_ref, w_ref, h_ref):
    h_ref[...] = jnp.dot(x_ref[...], w_ref[...],
                         preferred_element_type=jnp.float32)


def _scale_body(h_ref, hist_ref, g_ref):
    deg = jnp.sum(hist_ref[...], axis=1, keepdims=True) + 1.0
    g_ref[...] = h_ref[...] * lax.rsqrt(deg)


def _final_body(acc_ref, hist_ref, b_ref, out_ref):
    deg = jnp.sum(hist_ref[...], axis=1, keepdims=True) + 1.0
    s = lax.rsqrt(deg)
    out_ref[...] = (acc_ref[0] + acc_ref[1]) * s + b_ref[...]


def kernel(x, edge_index, W, b):
    n, d_in = x.shape
    d = W.shape[1]
    e = edge_index.shape[1]

    n_pad = (n // 128 + 1) * 128
    n_pad = ((n_pad + 2047) // 2048) * 2048
    e_pad = ((e + NW * CHUNK - 1) // (NW * CHUNK)) * (NW * CHUNK)
    cpt = e_pad // (NW * CHUNK)
    rpt = n_pad // NS
    pad_cnt = e_pad - e
    pad_rows = n_pad - n

    pad_idx = (n + jnp.arange(pad_cnt, dtype=jnp.int32) % pad_rows)
    ei_pad = jnp.concatenate(
        [edge_index, jnp.stack([pad_idx, pad_idx])], axis=1)
    edges_p = ei_pad.reshape(2, NW, cpt, CHUNK).transpose(1, 2, 0, 3)

    hist = _make_hist(n_pad, cpt)(edges_p).T

    zrow_d = jnp.zeros((rpt, d), jnp.float32)
    bt = 2000
    grid = (n // bt,)
    h = pl.pallas_call(
        _matmul_body,
        grid=grid,
        in_specs=[
            pl.BlockSpec((bt, d_in), lambda i: (i, 0)),
            pl.BlockSpec((d_in, d), lambda i: (0, 0)),
        ],
        out_specs=pl.BlockSpec((bt, d), lambda i: (i, 0)),
        out_shape=jax.ShapeDtypeStruct((n, d), jnp.float32),
    )(x, W)
    g = pl.pallas_call(
        _scale_body,
        grid=grid,
        in_specs=[
            pl.BlockSpec((bt, d), lambda i: (i, 0)),
            pl.BlockSpec((bt, NW), lambda i: (i, 0)),
        ],
        out_specs=pl.BlockSpec((bt, d), lambda i: (i, 0)),
        out_shape=jax.ShapeDtypeStruct((n_pad, d), jnp.float32),
    )(h, hist)

    acc = _make_spmm(n_pad, cpt, d)(g, edges_p, zrow_d)

    out = pl.pallas_call(
        _final_body,
        grid=grid,
        in_specs=[
            pl.BlockSpec((NC, bt, d), lambda i: (0, i, 0)),
            pl.BlockSpec((bt, NW), lambda i: (i, 0)),
            pl.BlockSpec((1, d), lambda i: (0, 0)),
        ],
        out_specs=pl.BlockSpec((bt, d), lambda i: (i, 0)),
        out_shape=jax.ShapeDtypeStruct((n, d), jnp.float32),
    )(acc, hist, b.reshape(1, d))

    return out


# speedup vs baseline: 47.9003x; 1.0973x over previous
import functools

import jax
import jax.numpy as jnp
from jax import lax
from jax.experimental import pallas as pl
from jax.experimental.pallas import tpu as pltpu
from jax.experimental.pallas import tpu_sc as plsc

NC = 2
NS = 16
NW = NC * NS
CHUNK = 128


def _sc_mesh():
    return plsc.VectorSubcoreMesh(
        core_axis_name="c", subcore_axis_name="s", num_cores=NC,
        num_subcores=NS)


def _make_hist(n_pad, cpt):
    @functools.partial(
        pl.kernel,
        out_type=jax.ShapeDtypeStruct((NW, n_pad), jnp.float32),
        mesh=_sc_mesh(),
        compiler_params=pltpu.CompilerParams(needs_layout_passes=False),
        scratch_types=[
            pltpu.VMEM((cpt, 2, CHUNK), jnp.int32),
            pltpu.VMEM((n_pad,), jnp.float32),
        ],
    )
    def hist_kernel(edges_hbm, hist_out, ed_v, loc_v):
        cid = lax.axis_index("c")
        sid = lax.axis_index("s")
        wid = sid * NC + cid

        def zstep(i, carry):
            loc_v[pl.ds(i * 16, 16)] = jnp.zeros((16,), jnp.float32)
            return carry

        lax.fori_loop(0, n_pad // 16, zstep, 0)
        pltpu.sync_copy(edges_hbm.at[wid], ed_v)
        ones = jnp.ones((16,), jnp.float32)

        def step(j, carry):
            def inner(i, carry2):
                idx = ed_v[j, 1, pl.ds(i * 16, 16)]
                plsc.addupdate_scatter(loc_v, [idx], ones)
                return carry2

            return lax.fori_loop(0, CHUNK // 16, inner, carry)

        lax.fori_loop(0, cpt, step, 0)
        pltpu.sync_copy(loc_v, hist_out.at[wid])

    return hist_kernel


def _make_spmm(n_pad, cpt, d):
    rpt = n_pad // NS

    @functools.partial(
        pl.kernel,
        out_type=jax.ShapeDtypeStruct((NC, n_pad, d), jnp.float32),
        mesh=_sc_mesh(),
        scratch_types=[
            pltpu.VMEM((2, CHUNK), jnp.int32),
            pltpu.VMEM((2, CHUNK), jnp.int32),
            pltpu.VMEM((2, CHUNK), jnp.int32),
            pltpu.VMEM((2, CHUNK), jnp.int32),
            pltpu.VMEM((CHUNK, d), jnp.float32),
            pltpu.VMEM((CHUNK, d), jnp.float32),
            pltpu.VMEM_SHARED((n_pad, d), jnp.float32),
            pltpu.SemaphoreType.DMA,
            pltpu.SemaphoreType.DMA,
            pltpu.SemaphoreType.DMA,
            pltpu.SemaphoreType.DMA,
            pltpu.SemaphoreType.DMA,
            pltpu.SemaphoreType.DMA,
        ],
    )
    def spmm_kernel(g_hbm, edges_hbm, zrow_hbm, acc_out, win0_v, win1_v,
                    win2_v, win3_v, rows0_v, rows1_v, acc_sh,
                    semg0, semg1, semi0, semi1, semi2, semi3):
        cid = lax.axis_index("c")
        sid = lax.axis_index("s")
        wid = sid * NC + cid

        pltpu.sync_copy(edges_hbm.at[wid].at[0], win0_v)
        pltpu.async_copy(g_hbm.at[win0_v.at[0]], rows0_v, semg0)

        @pl.when(1 < cpt)
        def _():
            pltpu.async_copy(edges_hbm.at[wid].at[1], win1_v, semi1)

        @pl.when(2 < cpt)
        def _():
            pltpu.async_copy(edges_hbm.at[wid].at[2], win2_v, semi2)

        @pl.when(cid == 0)
        def _():
            pltpu.sync_copy(g_hbm.at[pl.ds(sid * rpt, rpt)],
                            acc_sh.at[pl.ds(sid * rpt, rpt)])

        @pl.when(cid == 1)
        def _():
            pltpu.sync_copy(zrow_hbm, acc_sh.at[pl.ds(sid * rpt, rpt)])

        plsc.subcore_barrier()

        def wait_rows(buf, sem):
            pltpu.make_async_copy(g_hbm.at[pl.ds(0, CHUNK)], buf, sem).wait()

        def wait_win(buf, sem):
            pltpu.make_async_copy(edges_hbm.at[0].at[0], buf, sem).wait()

        def step(k, carry):
            j0 = 4 * k
            j1 = j0 + 1
            j2 = j0 + 2
            j3 = j0 + 3
            j4 = j0 + 4
            j5 = j0 + 5
            j6 = j0 + 6

            @pl.when(j1 < cpt)
            def _():
                wait_win(win1_v, semi1)
                pltpu.async_copy(g_hbm.at[win1_v.at[0]], rows1_v, semg1)

            @pl.when(j3 < cpt)
            def _():
                pltpu.async_copy(edges_hbm.at[wid].at[j3], win3_v, semi3)

            wait_rows(rows0_v, semg0)
            pltpu.sync_copy(rows0_v, acc_sh.at[win0_v.at[1]], add=True)

            @pl.when(j2 < cpt)
            def _():
                wait_win(win2_v, semi2)
                pltpu.async_copy(g_hbm.at[win2_v.at[0]], rows0_v, semg0)

            @pl.when(j4 < cpt)
            def _():
                pltpu.async_copy(edges_hbm.at[wid].at[j4], win0_v, semi0)

            @pl.when(j1 < cpt)
            def _():
                wait_rows(rows1_v, semg1)
                pltpu.sync_copy(rows1_v, acc_sh.at[win1_v.at[1]], add=True)

            @pl.when(j3 < cpt)
            def _():
                wait_win(win3_v, semi3)
                pltpu.async_copy(g_hbm.at[win3_v.at[0]], rows1_v, semg1)

            @pl.when(j5 < cpt)
            def _():
                pltpu.async_copy(edges_hbm.at[wid].at[j5], win1_v, semi1)

            @pl.when(j2 < cpt)
            def _():
                wait_rows(rows0_v, semg0)
                pltpu.sync_copy(rows0_v, acc_sh.at[win2_v.at[1]], add=True)

            @pl.when(j6 < cpt)
            def _():
                pltpu.async_copy(edges_hbm.at[wid].at[j6], win2_v, semi2)

            @pl.when(j4 < cpt)
            def _():
                wait_win(win0_v, semi0)
                pltpu.async_copy(g_hbm.at[win0_v.at[0]], rows0_v, semg0)

            @pl.when(j3 < cpt)
            def _():
                wait_rows(rows1_v, semg1)
                pltpu.sync_copy(rows1_v, acc_sh.at[win3_v.at[1]], add=True)

            return carry

        lax.fori_loop(0, (cpt + 3) // 4, step, 0)
        plsc.subcore_barrier()
        pltpu.sync_copy(acc_sh.at[pl.ds(sid * rpt, rpt)],
                        acc_out.at[cid].at[pl.ds(sid * rpt, rpt)])

    return spmm_kernel


def _matmul_body(x_ref, w_ref, h_ref):
    h_ref[...] = jnp.dot(x_ref[...], w_ref[...],
                         preferred_element_type=jnp.float32)


def _scale_body(h_ref, hist_ref, g_ref):
    deg = jnp.sum(hist_ref[...], axis=1, keepdims=True) + 1.0
    g_ref[...] = h_ref[...] * lax.rsqrt(deg)


def _final_body(acc_ref, hist_ref, b_ref, out_ref):
    deg = jnp.sum(hist_ref[...], axis=1, keepdims=True) + 1.0
    s = lax.rsqrt(deg)
    out_ref[...] = (acc_ref[0] + acc_ref[1]) * s + b_ref[...]


def kernel(x, edge_index, W, b):
    n, d_in = x.shape
    d = W.shape[1]
    e = edge_index.shape[1]

    n_pad = (n // 128 + 1) * 128
    n_pad = ((n_pad + 2047) // 2048) * 2048
    e_pad = ((e + NW * CHUNK - 1) // (NW * CHUNK)) * (NW * CHUNK)
    cpt = e_pad // (NW * CHUNK)
    rpt = n_pad // NS
    pad_cnt = e_pad - e
    pad_rows = n_pad - n

    pad_idx = (n + jnp.arange(pad_cnt, dtype=jnp.int32) % pad_rows)
    ei_pad = jnp.concatenate(
        [edge_index, jnp.stack([pad_idx, pad_idx])], axis=1)
    edges_p = ei_pad.reshape(2, NW, cpt, CHUNK).transpose(1, 2, 0, 3)

    hist = _make_hist(n_pad, cpt)(edges_p).T

    zrow_d = jnp.zeros((rpt, d), jnp.float32)
    bt = 2000
    grid = (n // bt,)
    h = pl.pallas_call(
        _matmul_body,
        grid=grid,
        in_specs=[
            pl.BlockSpec((bt, d_in), lambda i: (i, 0)),
            pl.BlockSpec((d_in, d), lambda i: (0, 0)),
        ],
        out_specs=pl.BlockSpec((bt, d), lambda i: (i, 0)),
        out_shape=jax.ShapeDtypeStruct((n, d), jnp.float32),
    )(x, W)
    g = pl.pallas_call(
        _scale_body,
        grid=grid,
        in_specs=[
            pl.BlockSpec((bt, d), lambda i: (i, 0)),
            pl.BlockSpec((bt, NW), lambda i: (i, 0)),
        ],
        out_specs=pl.BlockSpec((bt, d), lambda i: (i, 0)),
        out_shape=jax.ShapeDtypeStruct((n_pad, d), jnp.float32),
    )(h, hist)

    acc = _make_spmm(n_pad, cpt, d)(g, edges_p, zrow_d)

    out = pl.pallas_call(
        _final_body,
        grid=grid,
        in_specs=[
            pl.BlockSpec((NC, bt, d), lambda i: (0, i, 0)),
            pl.BlockSpec((bt, NW), lambda i: (i, 0)),
            pl.BlockSpec((1, d), lambda i: (0, 0)),
        ],
        out_specs=pl.BlockSpec((bt, d), lambda i: (i, 0)),
        out_shape=jax.ShapeDtypeStruct((n, d), jnp.float32),
    )(acc, hist, b.reshape(1, d))

    return out


# speedup vs baseline: 48.6592x; 1.0158x over previous
import functools

import jax
import jax.numpy as jnp
from jax import lax
from jax.experimental import pallas as pl
from jax.experimental.pallas import tpu as pltpu
from jax.experimental.pallas import tpu_sc as plsc

NC = 2
NS = 16
NW = NC * NS
CHUNK = 128


def _sc_mesh():
    return plsc.VectorSubcoreMesh(
        core_axis_name="c", subcore_axis_name="s", num_cores=NC,
        num_subcores=NS)


def _make_hist(n_pad, cpt):
    @functools.partial(
        pl.kernel,
        out_type=jax.ShapeDtypeStruct((NW, n_pad), jnp.float32),
        mesh=_sc_mesh(),
        compiler_params=pltpu.CompilerParams(needs_layout_passes=False),
        scratch_types=[
            pltpu.VMEM((cpt, 2, CHUNK), jnp.int32),
            pltpu.VMEM((n_pad,), jnp.float32),
        ],
    )
    def hist_kernel(edges_hbm, hist_out, ed_v, loc_v):
        cid = lax.axis_index("c")
        sid = lax.axis_index("s")
        wid = sid * NC + cid

        def zstep(i, carry):
            z = jnp.zeros((16,), jnp.float32)
            for u in range(8):
                loc_v[pl.ds(i * 128 + u * 16, 16)] = z
            return carry

        lax.fori_loop(0, n_pad // 128, zstep, 0)
        pltpu.sync_copy(edges_hbm.at[wid], ed_v)
        ones = jnp.ones((16,), jnp.float32)

        def step(j, carry):
            for i in range(CHUNK // 16):
                idx = ed_v[j, 1, pl.ds(i * 16, 16)]
                plsc.addupdate_scatter(loc_v, [idx], ones)
            return carry

        lax.fori_loop(0, cpt, step, 0)
        pltpu.sync_copy(loc_v, hist_out.at[wid])

    return hist_kernel


def _make_spmm(n_pad, cpt, d):
    rpt = n_pad // NS

    @functools.partial(
        pl.kernel,
        out_type=jax.ShapeDtypeStruct((NC, n_pad, d), jnp.float32),
        mesh=_sc_mesh(),
        scratch_types=[
            pltpu.VMEM((2, CHUNK), jnp.int32),
            pltpu.VMEM((2, CHUNK), jnp.int32),
            pltpu.VMEM((2, CHUNK), jnp.int32),
            pltpu.VMEM((2, CHUNK), jnp.int32),
            pltpu.VMEM((CHUNK, d), jnp.float32),
            pltpu.VMEM((CHUNK, d), jnp.float32),
            pltpu.VMEM_SHARED((n_pad, d), jnp.float32),
            pltpu.SemaphoreType.DMA,
            pltpu.SemaphoreType.DMA,
            pltpu.SemaphoreType.DMA,
            pltpu.SemaphoreType.DMA,
            pltpu.SemaphoreType.DMA,
            pltpu.SemaphoreType.DMA,
        ],
    )
    def spmm_kernel(g_hbm, edges_hbm, zrow_hbm, acc_out, win0_v, win1_v,
                    win2_v, win3_v, rows0_v, rows1_v, acc_sh,
                    semg0, semg1, semi0, semi1, semi2, semi3):
        cid = lax.axis_index("c")
        sid = lax.axis_index("s")
        wid = sid * NC + cid

        pltpu.sync_copy(edges_hbm.at[wid].at[0], win0_v)
        pltpu.async_copy(g_hbm.at[win0_v.at[0]], rows0_v, semg0)

        @pl.when(1 < cpt)
        def _():
            pltpu.async_copy(edges_hbm.at[wid].at[1], win1_v, semi1)

        @pl.when(2 < cpt)
        def _():
            pltpu.async_copy(edges_hbm.at[wid].at[2], win2_v, semi2)

        @pl.when(cid == 0)
        def _():
            pltpu.sync_copy(g_hbm.at[pl.ds(sid * rpt, rpt)],
                            acc_sh.at[pl.ds(sid * rpt, rpt)])

        @pl.when(cid == 1)
        def _():
            pltpu.sync_copy(zrow_hbm, acc_sh.at[pl.ds(sid * rpt, rpt)])

        plsc.subcore_barrier()

        def wait_rows(buf, sem):
            pltpu.make_async_copy(g_hbm.at[pl.ds(0, CHUNK)], buf, sem).wait()

        def wait_win(buf, sem):
            pltpu.make_async_copy(edges_hbm.at[0].at[0], buf, sem).wait()

        def step(k, carry):
            j0 = 4 * k
            j1 = j0 + 1
            j2 = j0 + 2
            j3 = j0 + 3
            j4 = j0 + 4
            j5 = j0 + 5
            j6 = j0 + 6

            @pl.when(j1 < cpt)
            def _():
                wait_win(win1_v, semi1)
                pltpu.async_copy(g_hbm.at[win1_v.at[0]], rows1_v, semg1)

            @pl.when(j3 < cpt)
            def _():
                pltpu.async_copy(edges_hbm.at[wid].at[j3], win3_v, semi3)

            wait_rows(rows0_v, semg0)
            pltpu.sync_copy(rows0_v, acc_sh.at[win0_v.at[1]], add=True)

            @pl.when(j2 < cpt)
            def _():
                wait_win(win2_v, semi2)
                pltpu.async_copy(g_hbm.at[win2_v.at[0]], rows0_v, semg0)

            @pl.when(j4 < cpt)
            def _():
                pltpu.async_copy(edges_hbm.at[wid].at[j4], win0_v, semi0)

            @pl.when(j1 < cpt)
            def _():
                wait_rows(rows1_v, semg1)
                pltpu.sync_copy(rows1_v, acc_sh.at[win1_v.at[1]], add=True)

            @pl.when(j3 < cpt)
            def _():
                wait_win(win3_v, semi3)
                pltpu.async_copy(g_hbm.at[win3_v.at[0]], rows1_v, semg1)

            @pl.when(j5 < cpt)
            def _():
                pltpu.async_copy(edges_hbm.at[wid].at[j5], win1_v, semi1)

            @pl.when(j2 < cpt)
            def _():
                wait_rows(rows0_v, semg0)
                pltpu.sync_copy(rows0_v, acc_sh.at[win2_v.at[1]], add=True)

            @pl.when(j6 < cpt)
            def _():
                pltpu.async_copy(edges_hbm.at[wid].at[j6], win2_v, semi2)

            @pl.when(j4 < cpt)
            def _():
                wait_win(win0_v, semi0)
                pltpu.async_copy(g_hbm.at[win0_v.at[0]], rows0_v, semg0)

            @pl.when(j3 < cpt)
            def _():
                wait_rows(rows1_v, semg1)
                pltpu.sync_copy(rows1_v, acc_sh.at[win3_v.at[1]], add=True)

            return carry

        lax.fori_loop(0, (cpt + 3) // 4, step, 0)
        plsc.subcore_barrier()
        pltpu.sync_copy(acc_sh.at[pl.ds(sid * rpt, rpt)],
                        acc_out.at[cid].at[pl.ds(sid * rpt, rpt)])

    return spmm_kernel


def _matmul_body(x_ref, w_ref, h_ref):
    h_ref[...] = jnp.dot(x_ref[...], w_ref[...],
                         preferred_element_type=jnp.float32)


def _scale_body(h_ref, hist_ref, g_ref):
    deg = jnp.sum(hist_ref[...], axis=1, keepdims=True) + 1.0
    g_ref[...] = h_ref[...] * lax.rsqrt(deg)


def _final_body(acc_ref, hist_ref, b_ref, out_ref):
    deg = jnp.sum(hist_ref[...], axis=1, keepdims=True) + 1.0
    s = lax.rsqrt(deg)
    out_ref[...] = (acc_ref[0] + acc_ref[1]) * s + b_ref[...]


def kernel(x, edge_index, W, b):
    n, d_in = x.shape
    d = W.shape[1]
    e = edge_index.shape[1]

    n_pad = (n // 128 + 1) * 128
    n_pad = ((n_pad + 2047) // 2048) * 2048
    e_pad = ((e + NW * CHUNK - 1) // (NW * CHUNK)) * (NW * CHUNK)
    cpt = e_pad // (NW * CHUNK)
    rpt = n_pad // NS
    pad_cnt = e_pad - e
    pad_rows = n_pad - n

    pad_idx = (n + jnp.arange(pad_cnt, dtype=jnp.int32) % pad_rows)
    ei_pad = jnp.concatenate(
        [edge_index, jnp.stack([pad_idx, pad_idx])], axis=1)
    edges_p = ei_pad.reshape(2, NW, cpt, CHUNK).transpose(1, 2, 0, 3)

    hist = _make_hist(n_pad, cpt)(edges_p).T

    zrow_d = jnp.zeros((rpt, d), jnp.float32)
    bt = 2000
    grid = (n // bt,)
    h = pl.pallas_call(
        _matmul_body,
        grid=grid,
        in_specs=[
            pl.BlockSpec((bt, d_in), lambda i: (i, 0)),
            pl.BlockSpec((d_in, d), lambda i: (0, 0)),
        ],
        out_specs=pl.BlockSpec((bt, d), lambda i: (i, 0)),
        out_shape=jax.ShapeDtypeStruct((n, d), jnp.float32),
    )(x, W)
    g = pl.pallas_call(
        _scale_body,
        grid=grid,
        in_specs=[
            pl.BlockSpec((bt, d), lambda i: (i, 0)),
            pl.BlockSpec((bt, NW), lambda i: (i, 0)),
        ],
        out_specs=pl.BlockSpec((bt, d), lambda i: (i, 0)),
        out_shape=jax.ShapeDtypeStruct((n_pad, d), jnp.float32),
    )(h, hist)

    acc = _make_spmm(n_pad, cpt, d)(g, edges_p, zrow_d)

    out = pl.pallas_call(
        _final_body,
        grid=grid,
        in_specs=[
            pl.BlockSpec((NC, bt, d), lambda i: (0, i, 0)),
            pl.BlockSpec((bt, NW), lambda i: (i, 0)),
            pl.BlockSpec((1, d), lambda i: (0, 0)),
        ],
        out_specs=pl.BlockSpec((bt, d), lambda i: (i, 0)),
        out_shape=jax.ShapeDtypeStruct((n, d), jnp.float32),
    )(acc, hist, b.reshape(1, d))

    return out
